# baseline (speedup 1.0000x reference)
import math

import jax
import jax.numpy as jnp
from jax.experimental import pallas as pl
from jax.experimental.pallas import tpu as pltpu

_BF16 = jnp.bfloat16
_N_LAYERS = 4
_N_HEADS = 8


def _bf(a):
    return a.astype(_BF16)


def _layernorm(y, g, b):
    mu = jnp.mean(y, axis=-1, keepdims=True)
    var = jnp.mean((y - mu) ** 2, axis=-1, keepdims=True)
    return (y - mu) * jax.lax.rsqrt(var + 1e-5) * g + b


def _make_fused_kernel(S, D, H, d_fc):
    dh = D // H
    scale = 1.0 / math.sqrt(dh)

    def _body(*refs):
        x_ref = refs[0]
        lrefs = refs[1:1 + 12 * _N_LAYERS]
        mlm_w_ref = refs[1 + 12 * _N_LAYERS]
        mlm_b_ref = refs[2 + 12 * _N_LAYERS]
        o_mlm_ref = refs[3 + 12 * _N_LAYERS]
        o_x_ref = refs[4 + 12 * _N_LAYERS]

        x = x_ref[0]
        for l in range(_N_LAYERS):
            (wqkv, bqkv, wo, bo, ln1g, ln1b,
             w1, b1, w2, b2, ln2g, ln2b) = lrefs[12 * l:12 * (l + 1)]

            qkv = (jnp.dot(_bf(x), wqkv[...],
                           preferred_element_type=jnp.float32) + bqkv[...])

            heads = []
            for h in range(H):
                qh = qkv[:, h * dh:(h + 1) * dh]
                kh = qkv[:, D + h * dh:D + (h + 1) * dh]
                vh = qkv[:, 2 * D + h * dh:2 * D + (h + 1) * dh]
                s = jax.lax.dot_general(
                    _bf(qh), _bf(kh), (((1,), (1,)), ((), ())),
                    preferred_element_type=jnp.float32) * scale
                p = jnp.exp(s - jnp.max(s, axis=-1, keepdims=True))
                p = p * pl.reciprocal(jnp.sum(p, axis=-1, keepdims=True),
                                      approx=True)
                heads.append(jnp.dot(_bf(p), _bf(vh),
                                     preferred_element_type=jnp.float32))
            attn = jnp.concatenate(heads, axis=-1)

            proj = (jnp.dot(_bf(attn), wo[...],
                            preferred_element_type=jnp.float32) + bo[...])
            x1 = _layernorm(proj + x, ln1g[...], ln1b[...])

            hdn = jnp.maximum(
                jnp.dot(_bf(x1), w1[...],
                        preferred_element_type=jnp.float32) + b1[...], 0.0)
            y2 = (jnp.dot(_bf(hdn), w2[...],
                          preferred_element_type=jnp.float32) + b2[...]) + x1
            x = _layernorm(y2, ln2g[...], ln2b[...])

        o_x_ref[0] = x
        o_mlm_ref[0] = (jnp.dot(_bf(x), mlm_w_ref[...],
                                preferred_element_type=jnp.float32)
                        + mlm_b_ref[...])

    return _body


def _encoder_and_mlm(x, layer_args, mlm_w, mlm_b, n_heads):
    B, S, D = x.shape
    d_fc = layer_args[6].shape[1]
    Npad = mlm_w.shape[1]

    def wspec(shape):
        nd = len(shape)
        return pl.BlockSpec(shape, lambda b: (0,) * nd)

    in_specs = [pl.BlockSpec((1, S, D), lambda b: (b, 0, 0))]
    ops = [x]
    for i, a in enumerate(layer_args):
        in_specs.append(wspec(a.shape))
        ops.append(a)
    in_specs += [wspec(mlm_w.shape), wspec(mlm_b.shape)]
    ops += [mlm_w, mlm_b]

    return pl.pallas_call(
        _make_fused_kernel(S, D, n_heads, d_fc),
        out_shape=[jax.ShapeDtypeStruct((B, S, Npad), jnp.float32),
                   jax.ShapeDtypeStruct((B, S, D), jnp.float32)],
        grid=(B,),
        in_specs=in_specs,
        out_specs=[pl.BlockSpec((1, S, Npad), lambda b: (b, 0, 0)),
                   pl.BlockSpec((1, S, D), lambda b: (b, 0, 0))],
        compiler_params=pltpu.CompilerParams(
            dimension_semantics=("parallel",)),
    )(*ops)


def kernel(x_ids, token_emb, pos_emb, mlm_w_pad, mlm_b_pad, cls_w, cls_b,
           l0_wqkv, l0_bqkv, l0_wo, l0_bo, l0_w1, l0_b1, l0_w2, l0_b2,
           l0_ln1_g, l0_ln1_b, l0_ln2_g, l0_ln2_b,
           l1_wqkv, l1_bqkv, l1_wo, l1_bo, l1_w1, l1_b1, l1_w2, l1_b2,
           l1_ln1_g, l1_ln1_b, l1_ln2_g, l1_ln2_b,
           l2_wqkv, l2_bqkv, l2_wo, l2_bo, l2_w1, l2_b1, l2_w2, l2_b2,
           l2_ln1_g, l2_ln1_b, l2_ln2_g, l2_ln2_b,
           l3_wqkv, l3_bqkv, l3_wo, l3_bo, l3_w1, l3_b1, l3_w2, l3_b2,
           l3_ln1_g, l3_ln1_b, l3_ln2_g, l3_ln2_b):
    B, S = x_ids.shape
    mlm_cls_num = token_emb.shape[0]

    x = jnp.take(token_emb, x_ids, axis=0) + pos_emb[:S][None, :, :]

    layers = [
        (l0_wqkv, l0_bqkv, l0_wo, l0_bo, l0_ln1_g, l0_ln1_b,
         l0_w1, l0_b1, l0_w2, l0_b2, l0_ln2_g, l0_ln2_b),
        (l1_wqkv, l1_bqkv, l1_wo, l1_bo, l1_ln1_g, l1_ln1_b,
         l1_w1, l1_b1, l1_w2, l1_b2, l1_ln2_g, l1_ln2_b),
        (l2_wqkv, l2_bqkv, l2_wo, l2_bo, l2_ln1_g, l2_ln1_b,
         l2_w1, l2_b1, l2_w2, l2_b2, l2_ln2_g, l2_ln2_b),
        (l3_wqkv, l3_bqkv, l3_wo, l3_bo, l3_ln1_g, l3_ln1_b,
         l3_w1, l3_b1, l3_w2, l3_b2, l3_ln2_g, l3_ln2_b),
    ]
    layer_args = []
    for (wqkv, bqkv, wo, bo, g1, b1n, w1, b1, w2, b2, g2, b2n) in layers:
        layer_args += [_bf(wqkv), bqkv, _bf(wo), bo, g1, b1n,
                       _bf(w1), b1, _bf(w2), b2, g2, b2n]

    logits_pad, x_out = _encoder_and_mlm(
        x, layer_args, _bf(mlm_w_pad), mlm_b_pad, _N_HEADS)

    yp_mlm = logits_pad[:, :, :mlm_cls_num]
    yp_cls = jnp.dot(x_out[:, 0, :], cls_w) + cls_b
    return yp_mlm, yp_cls


# speedup vs baseline: 1.7433x; 1.7433x over previous
import math

import jax
import jax.numpy as jnp
from jax.experimental import pallas as pl
from jax.experimental.pallas import tpu as pltpu

_BF16 = jnp.bfloat16
_N_LAYERS = 4
_N_HEADS = 8


def _bf(a):
    return a.astype(_BF16)


def _layernorm(y, g, b):
    mu = jnp.mean(y, axis=-1, keepdims=True)
    var = jnp.mean((y - mu) ** 2, axis=-1, keepdims=True)
    return (y - mu) * jax.lax.rsqrt(var + 1e-5) * g + b


def _make_fused_kernel(S, D, H, d_fc, n_batch):
    dh = D // H
    scale = 1.0 / math.sqrt(dh)
    M = n_batch * S

    def _body(*refs):
        x_ref = refs[0]
        lrefs = refs[1:1 + 12 * _N_LAYERS]
        mlm_w_ref = refs[1 + 12 * _N_LAYERS]
        mlm_b_ref = refs[2 + 12 * _N_LAYERS]
        o_mlm_ref = refs[3 + 12 * _N_LAYERS]
        o_x_ref = refs[4 + 12 * _N_LAYERS]

        x = x_ref[...].reshape(M, D)
        for l in range(_N_LAYERS):
            (wqkv, bqkv, wo, bo, ln1g, ln1b,
             w1, b1, w2, b2, ln2g, ln2b) = lrefs[12 * l:12 * (l + 1)]

            qkv = (jnp.dot(_bf(x), wqkv[...],
                           preferred_element_type=jnp.float32) + bqkv[...])

            scores = []
            vhs = []
            for b in range(n_batch):
                rows = slice(b * S, (b + 1) * S)
                for h in range(H):
                    qh = qkv[rows, h * dh:(h + 1) * dh] * scale
                    kh = qkv[rows, D + h * dh:D + (h + 1) * dh]
                    vhs.append(qkv[rows, 2 * D + h * dh:2 * D + (h + 1) * dh])
                    scores.append(jax.lax.dot_general(
                        _bf(qh), _bf(kh), (((1,), (1,)), ((), ())),
                        preferred_element_type=jnp.float32))
            outs = []
            prev_p = None
            for i, s in enumerate(scores):
                p = jnp.exp(s - jnp.max(s, axis=-1, keepdims=True))
                p = p * pl.reciprocal(jnp.sum(p, axis=-1, keepdims=True),
                                      approx=True)
                if prev_p is not None:
                    outs.append(jnp.dot(_bf(prev_p), _bf(vhs[i - 1]),
                                        preferred_element_type=jnp.float32))
                prev_p = p
            outs.append(jnp.dot(_bf(prev_p), _bf(vhs[-1]),
                                preferred_element_type=jnp.float32))
            attn = jnp.concatenate(
                [jnp.concatenate(outs[b * H:(b + 1) * H], axis=-1)
                 for b in range(n_batch)], axis=0)

            proj = (jnp.dot(_bf(attn), wo[...],
                            preferred_element_type=jnp.float32) + bo[...])
            x1 = _layernorm(proj + x, ln1g[...], ln1b[...])

            hdn = jnp.maximum(
                jnp.dot(_bf(x1), w1[...],
                        preferred_element_type=jnp.float32) + b1[...], 0.0)
            y2 = (jnp.dot(_bf(hdn), w2[...],
                          preferred_element_type=jnp.float32) + b2[...]) + x1
            x = _layernorm(y2, ln2g[...], ln2b[...])

        o_x_ref[...] = x.reshape(n_batch, S, D)
        logits = (jnp.dot(_bf(x), mlm_w_ref[...],
                          preferred_element_type=jnp.float32) + mlm_b_ref[...])
        o_mlm_ref[...] = logits.reshape(n_batch, S, -1)

    return _body


def _encoder_and_mlm(x, layer_args, mlm_w, mlm_b, n_heads):
    B, S, D = x.shape
    d_fc = layer_args[6].shape[1]
    Npad = mlm_w.shape[1]
    n_prog = 2
    nb = B // n_prog

    def wspec(shape):
        nd = len(shape)
        return pl.BlockSpec(shape, lambda b: (0,) * nd)

    in_specs = [pl.BlockSpec((nb, S, D), lambda b: (b, 0, 0))]
    ops = [x]
    for a in layer_args:
        in_specs.append(wspec(a.shape))
        ops.append(a)
    in_specs += [wspec(mlm_w.shape), wspec(mlm_b.shape)]
    ops += [mlm_w, mlm_b]

    return pl.pallas_call(
        _make_fused_kernel(S, D, n_heads, d_fc, nb),
        out_shape=[jax.ShapeDtypeStruct((B, S, Npad), jnp.float32),
                   jax.ShapeDtypeStruct((B, S, D), jnp.float32)],
        grid=(n_prog,),
        in_specs=in_specs,
        out_specs=[pl.BlockSpec((nb, S, Npad), lambda b: (b, 0, 0)),
                   pl.BlockSpec((nb, S, D), lambda b: (b, 0, 0))],
        compiler_params=pltpu.CompilerParams(
            dimension_semantics=("parallel",)),
    )(*ops)


def kernel(x_ids, token_emb, pos_emb, mlm_w_pad, mlm_b_pad, cls_w, cls_b,
           l0_wqkv, l0_bqkv, l0_wo, l0_bo, l0_w1, l0_b1, l0_w2, l0_b2,
           l0_ln1_g, l0_ln1_b, l0_ln2_g, l0_ln2_b,
           l1_wqkv, l1_bqkv, l1_wo, l1_bo, l1_w1, l1_b1, l1_w2, l1_b2,
           l1_ln1_g, l1_ln1_b, l1_ln2_g, l1_ln2_b,
           l2_wqkv, l2_bqkv, l2_wo, l2_bo, l2_w1, l2_b1, l2_w2, l2_b2,
           l2_ln1_g, l2_ln1_b, l2_ln2_g, l2_ln2_b,
           l3_wqkv, l3_bqkv, l3_wo, l3_bo, l3_w1, l3_b1, l3_w2, l3_b2,
           l3_ln1_g, l3_ln1_b, l3_ln2_g, l3_ln2_b):
    B, S = x_ids.shape
    mlm_cls_num = token_emb.shape[0]

    x = jnp.take(token_emb, x_ids, axis=0) + pos_emb[:S][None, :, :]

    layers = [
        (l0_wqkv, l0_bqkv, l0_wo, l0_bo, l0_ln1_g, l0_ln1_b,
         l0_w1, l0_b1, l0_w2, l0_b2, l0_ln2_g, l0_ln2_b),
        (l1_wqkv, l1_bqkv, l1_wo, l1_bo, l1_ln1_g, l1_ln1_b,
         l1_w1, l1_b1, l1_w2, l1_b2, l1_ln2_g, l1_ln2_b),
        (l2_wqkv, l2_bqkv, l2_wo, l2_bo, l2_ln1_g, l2_ln1_b,
         l2_w1, l2_b1, l2_w2, l2_b2, l2_ln2_g, l2_ln2_b),
        (l3_wqkv, l3_bqkv, l3_wo, l3_bo, l3_ln1_g, l3_ln1_b,
         l3_w1, l3_b1, l3_w2, l3_b2, l3_ln2_g, l3_ln2_b),
    ]
    layer_args = []
    for (wqkv, bqkv, wo, bo, g1, b1n, w1, b1, w2, b2, g2, b2n) in layers:
        layer_args += [_bf(wqkv), bqkv, _bf(wo), bo, g1, b1n,
                       _bf(w1), b1, _bf(w2), b2, g2, b2n]

    logits_pad, x_out = _encoder_and_mlm(
        x, layer_args, _bf(mlm_w_pad), mlm_b_pad, _N_HEADS)

    yp_mlm = logits_pad[:, :, :mlm_cls_num]
    yp_cls = jnp.dot(x_out[:, 0, :], cls_w) + cls_b
    return yp_mlm, yp_cls
